# baseline (speedup 1.0000x reference)
import functools

import jax
import jax.numpy as jnp
from jax import lax
from jax.experimental import pallas as pl
from jax.experimental.pallas import tpu as pltpu
from jax.experimental.pallas import tpu_sc as plsc

_CH = 80
_NW = 32
_NT = 16


def _make_sc_gather(N, D, E, RW):
    mesh = plsc.VectorSubcoreMesh(core_axis_name="c", subcore_axis_name="s")

    @functools.partial(
        pl.kernel, mesh=mesh,
        out_type=(
            jax.ShapeDtypeStruct((E, D), jnp.int32),
            jax.ShapeDtypeStruct((E, D), jnp.int32),
        ),
        scratch_types=[
            pltpu.VMEM((RW, _CH), jnp.int32),
            pltpu.VMEM((RW, _CH), jnp.int32),
            pltpu.VMEM((_CH, D), jnp.int32),
            pltpu.VMEM((_CH, D), jnp.int32),
            pltpu.SemaphoreType.DMA,
            pltpu.SemaphoreType.DMA,
        ],
    )
    def gather_k(table_h, sidx_h, didx_h, osrc_h, odst_h,
                 sidx_v, didx_v, sbuf_v, dbuf_v, sem0, sem1):
        wid = lax.axis_index("s") * 2 + lax.axis_index("c")
        pltpu.sync_copy(sidx_h.at[wid], sidx_v)
        pltpu.sync_copy(didx_h.at[wid], didx_v)

        def step(i, carry):
            base = wid * (RW * _CH) + i * _CH
            cps = pltpu.async_copy(table_h.at[sidx_v.at[i]], sbuf_v, sem0)
            cpd = pltpu.async_copy(table_h.at[didx_v.at[i]], dbuf_v, sem1)
            cps.wait()
            pltpu.sync_copy(sbuf_v, osrc_h.at[pl.ds(base, _CH)])
            cpd.wait()
            pltpu.sync_copy(dbuf_v, odst_h.at[pl.ds(base, _CH)])
            return carry

        lax.fori_loop(0, RW, step, 0)

    return gather_k


def _make_sc_scatter(NP, D, E, RW, first):
    mesh = plsc.VectorSubcoreMesh(core_axis_name="c", subcore_axis_name="s")
    RPT = NP // _NT

    @functools.partial(
        pl.kernel, mesh=mesh,
        out_type=jax.ShapeDtypeStruct((2, NP, D), jnp.float32),
        scratch_types=[
            pltpu.VMEM_SHARED((NP, D), jnp.float32),
            pltpu.VMEM((RW, _CH), jnp.int32),
            pltpu.VMEM((_CH, D), jnp.float32),
        ],
    )
    def scatter_k(raw_h, didx_h, init_h, out_h, acc_sh, didx_v, row_v):
        cid = lax.axis_index("c")
        sid = lax.axis_index("s")
        wid = sid * 2 + cid
        if first:
            pltpu.sync_copy(init_h, acc_sh.at[pl.ds(sid * RPT, RPT)])
        else:
            pltpu.sync_copy(init_h.at[cid, pl.ds(sid * RPT, RPT)],
                            acc_sh.at[pl.ds(sid * RPT, RPT)])
        plsc.subcore_barrier()
        pltpu.sync_copy(didx_h.at[wid], didx_v)

        def step(i, carry):
            base = wid * (RW * _CH) + i * _CH
            pltpu.sync_copy(raw_h.at[pl.ds(base, _CH)], row_v)
            pltpu.sync_copy(row_v, acc_sh.at[didx_v.at[i]], add=True)
            return carry

        lax.fori_loop(0, RW, step, 0)
        plsc.subcore_barrier()
        pltpu.sync_copy(acc_sh.at[pl.ds(sid * RPT, RPT)],
                        out_h.at[cid, pl.ds(sid * RPT, RPT)])

    return scatter_k


def _edge_body(*args):
    (ef, eh, sc_, dc, wrz_r, wxn_r, whn_r, wm_r, brz_r, bxn_r, bhn_r, bm_r,
     g_r, b_r) = args[:14]
    raw_o, nrm_o = args[-2:]
    D = ef.shape[-1]
    H = 3 * D
    sv = sc_[...]
    dv = dc[...]

    def unpack(vi):
        f = lax.bitcast_convert_type(vi << 16, jnp.float32)
        h = lax.bitcast_convert_type(
            vi & jnp.int32(-65536), jnp.float32)
        return f.astype(jnp.bfloat16), h.astype(jnp.bfloat16)

    sf, sh = unpack(sv)
    df, dh = unpack(dv)
    x12 = jnp.concatenate(
        [ef[...].astype(jnp.bfloat16), sf, df,
         eh[...].astype(jnp.bfloat16), sh, dh], axis=1)
    x2 = x12[:, H:]
    g_rz = jnp.dot(x12, wrz_r[...],
                   preferred_element_type=jnp.float32) + brz_r[...]
    i_n = jnp.dot(x12[:, :H], wxn_r[...],
                  preferred_element_type=jnp.float32) + bxn_r[...]
    h_n = jnp.dot(x2, whn_r[...],
                  preferred_element_type=jnp.float32) + bhn_r[...]
    r = jax.nn.sigmoid(g_rz[:, :H])
    z = jax.nn.sigmoid(g_rz[:, H:])
    n = jnp.tanh(i_n + r * h_n)
    hn = (1.0 - z) * n + z * x2.astype(jnp.float32)
    y = jnp.dot(hn.astype(jnp.bfloat16), wm_r[...],
                preferred_element_type=jnp.float32) + bm_r[...]
    raw_o[...] = y
    m = jnp.mean(y, axis=1, keepdims=True)
    v = jnp.mean((y - m) * (y - m), axis=1, keepdims=True)
    nrm_o[...] = (y - m) * lax.rsqrt(v + 1e-5) * g_r[...] + b_r[...]


def _make_node_body(nparts):
    def _node_body(*args):
        parts = args[:nparts]
        (nf, nh, wx_r, wh_r, wm_r, bx_r, bh_r, bm_r, g_r, b_r,
         out_o) = args[nparts:]
        _node_core(parts, nf, nh, wx_r, wh_r, wm_r, bx_r, bh_r, bm_r,
                   g_r, b_r, out_o)
    return _node_body


def _node_core(parts, nf, nh, wx_r, wh_r, wm_r, bx_r, bh_r, bm_r, g_r, b_r,
               out_o):
    D = nf.shape[-1]
    H = 2 * D
    agg = parts[0][0] + parts[0][1]
    for p in parts[1:]:
        agg = agg + p[0] + p[1]
    x1 = jnp.concatenate([agg, nf[...]], axis=1)
    x2 = jnp.concatenate([agg, nh[...]], axis=1)
    gx = jnp.dot(x1, wx_r[...], preferred_element_type=jnp.float32) + bx_r[...]
    gh = jnp.dot(x2, wh_r[...], preferred_element_type=jnp.float32) + bh_r[...]
    r = jax.nn.sigmoid(gx[:, :H] + gh[:, :H])
    z = jax.nn.sigmoid(gx[:, H:2 * H] + gh[:, H:2 * H])
    n = jnp.tanh(gx[:, 2 * H:] + r * gh[:, 2 * H:])
    hn = (1.0 - z) * n + z * x2
    y = jnp.dot(hn, wm_r[...], preferred_element_type=jnp.float32) + bm_r[...]
    m = jnp.mean(y, axis=1, keepdims=True)
    v = jnp.mean((y - m) * (y - m), axis=1, keepdims=True)
    out_o[...] = (y - m) * lax.rsqrt(v + 1e-5) * g_r[...] + b_r[...]


def _full(shape):
    return pl.BlockSpec(shape, lambda i: (0,) * len(shape))


def kernel(efeat, nfeat, ehidden, nhidden, edge_index,
           e_Wx, e_bx, e_Wh, e_bh, e_mlp_W, e_mlp_b,
           n_Wx, n_bx, n_Wh, n_bh, n_mlp_W, n_mlp_b,
           e_gamma, e_beta, n_gamma, n_beta):
    N, D = nfeat.shape
    E = efeat.shape[0]
    RWS = (5, 35, 40, 40, 5)
    SPAN = _NW * _CH

    src_r = edge_index[0].reshape(-1, _CH)
    dst_r = edge_index[1].reshape(-1, _CH)
    nf16 = lax.bitcast_convert_type(
        nfeat.astype(jnp.bfloat16), jnp.uint16).astype(jnp.uint32)
    nh16 = lax.bitcast_convert_type(
        nhidden.astype(jnp.bfloat16), jnp.uint16).astype(jnp.uint32)
    table = ((nh16 << 16) | nf16).astype(jnp.int32)

    BE = 2560
    H = 3 * D
    w_rz = jnp.concatenate(
        [e_Wx[:, :2 * H], e_Wh[:, :2 * H]], axis=0).astype(jnp.bfloat16)
    w_xn = e_Wx[:, 2 * H:].astype(jnp.bfloat16)
    w_hn = e_Wh[:, 2 * H:].astype(jnp.bfloat16)
    w_m = e_mlp_W.astype(jnp.bfloat16)
    eb = [b.reshape(1, -1) for b in
          (e_bx[:2 * H] + e_bh[:2 * H], e_bx[2 * H:], e_bh[2 * H:],
           e_mlp_b, e_gamma, e_beta)]
    NP = _NT * ((N + 8 * _NT - 1) // (8 * _NT)) * 8
    zeros_tile = jnp.zeros((NP // _NT, D), jnp.float32)

    gather_ks = {rw: _make_sc_gather(N, D, rw * SPAN, rw) for rw in set(RWS)}
    scatter_ks = {(rw, fst): _make_sc_scatter(NP, D, rw * SPAN, rw, fst)
                  for rw in set(RWS) for fst in (True, False)}

    def edge_chunk(off, rw, srcc, dstc, e_out_prev):
        ge = rw * SPAN // BE
        ob = off * SPAN // BE

        def blk(i_dim):
            return pl.BlockSpec((BE, i_dim), lambda i: (i + ob, 0))
        specs = [
            blk(D), blk(D),
            pl.BlockSpec((BE, D), lambda i: (i, 0)),
            pl.BlockSpec((BE, D), lambda i: (i, 0)),
            _full(w_rz.shape), _full(w_xn.shape), _full(w_hn.shape),
            _full(w_m.shape),
            _full(eb[0].shape), _full(eb[1].shape), _full(eb[2].shape),
            _full(eb[3].shape), _full(eb[4].shape), _full(eb[5].shape),
        ]
        ins = [efeat, ehidden, srcc, dstc, w_rz, w_xn, w_hn, w_m, *eb]
        aliases = {}
        if e_out_prev is not None:
            specs.append(pl.BlockSpec(memory_space=pl.ANY))
            ins.append(e_out_prev)
            aliases = {14: 1}
        return pl.pallas_call(
            _edge_body,
            grid=(ge,),
            in_specs=specs,
            out_specs=[
                pl.BlockSpec((BE, D), lambda i: (i, 0)),
                pl.BlockSpec((BE, D), lambda i: (i + ob, 0)),
            ],
            out_shape=[
                jax.ShapeDtypeStruct((rw * SPAN, D), jnp.float32),
                jax.ShapeDtypeStruct((E, D), jnp.float32),
            ],
            input_output_aliases=aliases,
            compiler_params=pltpu.CompilerParams(
                dimension_semantics=("arbitrary",)),
        )(*ins)

    e_out = None
    parts = zeros_tile
    off = 0
    for c, rw in enumerate(RWS):
        sidx = src_r[off * _NW:(off + rw) * _NW].reshape(_NW, rw, _CH)
        didx = dst_r[off * _NW:(off + rw) * _NW].reshape(_NW, rw, _CH)
        srcc, dstc = gather_ks[rw](table, sidx, didx)
        raw_c, e_out = edge_chunk(off, rw, srcc, dstc, e_out)
        parts = scatter_ks[(rw, c == 0)](raw_c, didx, parts)
        off += rw

    BN = 2000
    GN = N // BN
    nb = [b.reshape(1, -1) for b in (n_bx, n_bh, n_mlp_b, n_gamma, n_beta)]
    n_out = pl.pallas_call(
        _make_node_body(1),
        grid=(GN,),
        in_specs=[pl.BlockSpec((2, BN, D), lambda i: (0, i, 0))] + [
            pl.BlockSpec((BN, D), lambda i: (i, 0)),
            pl.BlockSpec((BN, D), lambda i: (i, 0)),
            _full(n_Wx.shape), _full(n_Wh.shape), _full(n_mlp_W.shape),
            _full(nb[0].shape), _full(nb[1].shape), _full(nb[2].shape),
            _full(nb[3].shape), _full(nb[4].shape),
        ],
        out_specs=pl.BlockSpec((BN, D), lambda i: (i, 0)),
        out_shape=jax.ShapeDtypeStruct((N, D), jnp.float32),
        compiler_params=pltpu.CompilerParams(
            dimension_semantics=("arbitrary",)),
    )(parts, nfeat, nhidden, n_Wx, n_Wh, n_mlp_W, *nb)

    return e_out, n_out


# speedup vs baseline: 1.0195x; 1.0195x over previous
import functools

import jax
import jax.numpy as jnp
from jax import lax
from jax.experimental import pallas as pl
from jax.experimental.pallas import tpu as pltpu
from jax.experimental.pallas import tpu_sc as plsc

_CH = 80
_NW = 32
_NT = 16


def _make_sc_gather(N, D, E, RW):
    mesh = plsc.VectorSubcoreMesh(core_axis_name="c", subcore_axis_name="s")

    @functools.partial(
        pl.kernel, mesh=mesh,
        out_type=(
            jax.ShapeDtypeStruct((E, D), jnp.int32),
            jax.ShapeDtypeStruct((E, D), jnp.int32),
        ),
        scratch_types=[
            pltpu.VMEM((RW, _CH), jnp.int32),
            pltpu.VMEM((RW, _CH), jnp.int32),
            pltpu.VMEM((_CH, D), jnp.int32),
            pltpu.VMEM((_CH, D), jnp.int32),
            pltpu.SemaphoreType.DMA,
            pltpu.SemaphoreType.DMA,
        ],
    )
    def gather_k(table_h, sidx_h, didx_h, osrc_h, odst_h,
                 sidx_v, didx_v, sbuf_v, dbuf_v, sem0, sem1):
        wid = lax.axis_index("s") * 2 + lax.axis_index("c")
        pltpu.sync_copy(sidx_h.at[wid], sidx_v)
        pltpu.sync_copy(didx_h.at[wid], didx_v)

        def step(i, carry):
            base = wid * (RW * _CH) + i * _CH
            cps = pltpu.async_copy(table_h.at[sidx_v.at[i]], sbuf_v, sem0)
            cpd = pltpu.async_copy(table_h.at[didx_v.at[i]], dbuf_v, sem1)
            cps.wait()
            pltpu.sync_copy(sbuf_v, osrc_h.at[pl.ds(base, _CH)])
            cpd.wait()
            pltpu.sync_copy(dbuf_v, odst_h.at[pl.ds(base, _CH)])
            return carry

        lax.fori_loop(0, RW, step, 0)

    return gather_k


def _make_sc_scatter(NP, D, E, RW, first):
    mesh = plsc.VectorSubcoreMesh(core_axis_name="c", subcore_axis_name="s")
    RPT = NP // _NT

    @functools.partial(
        pl.kernel, mesh=mesh,
        out_type=jax.ShapeDtypeStruct((2, NP, D), jnp.float32),
        scratch_types=[
            pltpu.VMEM_SHARED((NP, D), jnp.float32),
            pltpu.VMEM((RW, _CH), jnp.int32),
            pltpu.VMEM((_CH, D), jnp.float32),
        ],
    )
    def scatter_k(raw_h, didx_h, init_h, out_h, acc_sh, didx_v, row_v):
        cid = lax.axis_index("c")
        sid = lax.axis_index("s")
        wid = sid * 2 + cid
        if first:
            pltpu.sync_copy(init_h, acc_sh.at[pl.ds(sid * RPT, RPT)])
        else:
            pltpu.sync_copy(init_h.at[cid, pl.ds(sid * RPT, RPT)],
                            acc_sh.at[pl.ds(sid * RPT, RPT)])
        plsc.subcore_barrier()
        pltpu.sync_copy(didx_h.at[wid], didx_v)

        def step(i, carry):
            base = wid * (RW * _CH) + i * _CH
            pltpu.sync_copy(raw_h.at[pl.ds(base, _CH)], row_v)
            pltpu.sync_copy(row_v, acc_sh.at[didx_v.at[i]], add=True)
            return carry

        lax.fori_loop(0, RW, step, 0)
        plsc.subcore_barrier()
        pltpu.sync_copy(acc_sh.at[pl.ds(sid * RPT, RPT)],
                        out_h.at[cid, pl.ds(sid * RPT, RPT)])

    return scatter_k


def _edge_body(*args):
    (ef, eh, sc_, dc, wrz_r, wxn_r, whn_r, wm_r, brz_r, bxn_r, bhn_r, bm_r,
     g_r, b_r) = args[:14]
    raw_o, nrm_o = args[-2:]
    D = ef.shape[-1]
    H = 3 * D
    sv = sc_[...]
    dv = dc[...]

    def unpack(vi):
        f = lax.bitcast_convert_type(vi << 16, jnp.float32)
        h = lax.bitcast_convert_type(
            vi & jnp.int32(-65536), jnp.float32)
        return f.astype(jnp.bfloat16), h.astype(jnp.bfloat16)

    sf, sh = unpack(sv)
    df, dh = unpack(dv)
    x12 = jnp.concatenate(
        [ef[...].astype(jnp.bfloat16), sf, df,
         eh[...].astype(jnp.bfloat16), sh, dh], axis=1)
    x2 = x12[:, H:]
    g_rz = jnp.dot(x12, wrz_r[...],
                   preferred_element_type=jnp.float32) + brz_r[...]
    i_n = jnp.dot(x12[:, :H], wxn_r[...],
                  preferred_element_type=jnp.float32) + bxn_r[...]
    h_n = jnp.dot(x2, whn_r[...],
                  preferred_element_type=jnp.float32) + bhn_r[...]
    r = jax.nn.sigmoid(g_rz[:, :H])
    z = jax.nn.sigmoid(g_rz[:, H:])
    n = jnp.tanh(i_n + r * h_n)
    hn = (1.0 - z) * n + z * x2.astype(jnp.float32)
    y = jnp.dot(hn.astype(jnp.bfloat16), wm_r[...],
                preferred_element_type=jnp.float32) + bm_r[...]
    raw_o[...] = y
    m = jnp.mean(y, axis=1, keepdims=True)
    v = jnp.mean((y - m) * (y - m), axis=1, keepdims=True)
    nrm_o[...] = (y - m) * lax.rsqrt(v + 1e-5) * g_r[...] + b_r[...]


def _make_node_body(nparts):
    def _node_body(*args):
        parts = args[:nparts]
        (nf, nh, wx_r, wh_r, wm_r, bx_r, bh_r, bm_r, g_r, b_r,
         out_o) = args[nparts:]
        _node_core(parts, nf, nh, wx_r, wh_r, wm_r, bx_r, bh_r, bm_r,
                   g_r, b_r, out_o)
    return _node_body


def _node_core(parts, nf, nh, wx_r, wh_r, wm_r, bx_r, bh_r, bm_r, g_r, b_r,
               out_o):
    D = nf.shape[-1]
    H = 2 * D
    agg = parts[0][0] + parts[0][1]
    for p in parts[1:]:
        agg = agg + p[0] + p[1]
    x1 = jnp.concatenate([agg, nf[...]], axis=1)
    x2 = jnp.concatenate([agg, nh[...]], axis=1)
    gx = jnp.dot(x1, wx_r[...], preferred_element_type=jnp.float32) + bx_r[...]
    gh = jnp.dot(x2, wh_r[...], preferred_element_type=jnp.float32) + bh_r[...]
    r = jax.nn.sigmoid(gx[:, :H] + gh[:, :H])
    z = jax.nn.sigmoid(gx[:, H:2 * H] + gh[:, H:2 * H])
    n = jnp.tanh(gx[:, 2 * H:] + r * gh[:, 2 * H:])
    hn = (1.0 - z) * n + z * x2
    y = jnp.dot(hn, wm_r[...], preferred_element_type=jnp.float32) + bm_r[...]
    m = jnp.mean(y, axis=1, keepdims=True)
    v = jnp.mean((y - m) * (y - m), axis=1, keepdims=True)
    out_o[...] = (y - m) * lax.rsqrt(v + 1e-5) * g_r[...] + b_r[...]


def _full(shape):
    return pl.BlockSpec(shape, lambda i: (0,) * len(shape))


def kernel(efeat, nfeat, ehidden, nhidden, edge_index,
           e_Wx, e_bx, e_Wh, e_bh, e_mlp_W, e_mlp_b,
           n_Wx, n_bx, n_Wh, n_bh, n_mlp_W, n_mlp_b,
           e_gamma, e_beta, n_gamma, n_beta):
    N, D = nfeat.shape
    E = efeat.shape[0]
    RWS = (5, 35, 40, 35, 10)
    SPAN = _NW * _CH

    src_r = edge_index[0].reshape(-1, _CH)
    dst_r = edge_index[1].reshape(-1, _CH)
    nf16 = lax.bitcast_convert_type(
        nfeat.astype(jnp.bfloat16), jnp.uint16).astype(jnp.uint32)
    nh16 = lax.bitcast_convert_type(
        nhidden.astype(jnp.bfloat16), jnp.uint16).astype(jnp.uint32)
    table = ((nh16 << 16) | nf16).astype(jnp.int32)

    BE = 2560
    H = 3 * D
    w_rz = jnp.concatenate(
        [e_Wx[:, :2 * H], e_Wh[:, :2 * H]], axis=0).astype(jnp.bfloat16)
    w_xn = e_Wx[:, 2 * H:].astype(jnp.bfloat16)
    w_hn = e_Wh[:, 2 * H:].astype(jnp.bfloat16)
    w_m = e_mlp_W.astype(jnp.bfloat16)
    eb = [b.reshape(1, -1) for b in
          (e_bx[:2 * H] + e_bh[:2 * H], e_bx[2 * H:], e_bh[2 * H:],
           e_mlp_b, e_gamma, e_beta)]
    NP = _NT * ((N + 8 * _NT - 1) // (8 * _NT)) * 8
    zeros_tile = jnp.zeros((NP // _NT, D), jnp.float32)

    gather_ks = {rw: _make_sc_gather(N, D, rw * SPAN, rw) for rw in set(RWS)}
    scatter_ks = {(rw, fst): _make_sc_scatter(NP, D, rw * SPAN, rw, fst)
                  for rw in set(RWS) for fst in (True, False)}

    def edge_chunk(off, rw, srcc, dstc, e_out_prev):
        ge = rw * SPAN // BE
        ob = off * SPAN // BE

        def blk(i_dim):
            return pl.BlockSpec((BE, i_dim), lambda i: (i + ob, 0))
        specs = [
            blk(D), blk(D),
            pl.BlockSpec((BE, D), lambda i: (i, 0)),
            pl.BlockSpec((BE, D), lambda i: (i, 0)),
            _full(w_rz.shape), _full(w_xn.shape), _full(w_hn.shape),
            _full(w_m.shape),
            _full(eb[0].shape), _full(eb[1].shape), _full(eb[2].shape),
            _full(eb[3].shape), _full(eb[4].shape), _full(eb[5].shape),
        ]
        ins = [efeat, ehidden, srcc, dstc, w_rz, w_xn, w_hn, w_m, *eb]
        aliases = {}
        if e_out_prev is not None:
            specs.append(pl.BlockSpec(memory_space=pl.ANY))
            ins.append(e_out_prev)
            aliases = {14: 1}
        return pl.pallas_call(
            _edge_body,
            grid=(ge,),
            in_specs=specs,
            out_specs=[
                pl.BlockSpec((BE, D), lambda i: (i, 0)),
                pl.BlockSpec((BE, D), lambda i: (i + ob, 0)),
            ],
            out_shape=[
                jax.ShapeDtypeStruct((rw * SPAN, D), jnp.float32),
                jax.ShapeDtypeStruct((E, D), jnp.float32),
            ],
            input_output_aliases=aliases,
            compiler_params=pltpu.CompilerParams(
                dimension_semantics=("arbitrary",)),
        )(*ins)

    e_out = None
    parts = zeros_tile
    off = 0
    for c, rw in enumerate(RWS):
        sidx = src_r[off * _NW:(off + rw) * _NW].reshape(_NW, rw, _CH)
        didx = dst_r[off * _NW:(off + rw) * _NW].reshape(_NW, rw, _CH)
        srcc, dstc = gather_ks[rw](table, sidx, didx)
        raw_c, e_out = edge_chunk(off, rw, srcc, dstc, e_out)
        parts = scatter_ks[(rw, c == 0)](raw_c, didx, parts)
        off += rw

    BN = 2000
    GN = N // BN
    nb = [b.reshape(1, -1) for b in (n_bx, n_bh, n_mlp_b, n_gamma, n_beta)]
    n_out = pl.pallas_call(
        _make_node_body(1),
        grid=(GN,),
        in_specs=[pl.BlockSpec((2, BN, D), lambda i: (0, i, 0))] + [
            pl.BlockSpec((BN, D), lambda i: (i, 0)),
            pl.BlockSpec((BN, D), lambda i: (i, 0)),
            _full(n_Wx.shape), _full(n_Wh.shape), _full(n_mlp_W.shape),
            _full(nb[0].shape), _full(nb[1].shape), _full(nb[2].shape),
            _full(nb[3].shape), _full(nb[4].shape),
        ],
        out_specs=pl.BlockSpec((BN, D), lambda i: (i, 0)),
        out_shape=jax.ShapeDtypeStruct((N, D), jnp.float32),
        compiler_params=pltpu.CompilerParams(
            dimension_semantics=("arbitrary",)),
    )(parts, nfeat, nhidden, n_Wx, n_Wh, n_mlp_W, *nb)

    return e_out, n_out
